# baseline (speedup 1.0000x reference)
import functools

import jax
import jax.numpy as jnp
from jax import lax
from jax.experimental import pallas as pl
from jax.experimental.pallas import tpu as pltpu

_RB = 8192
_NBUF = 4
_TB = 2048


def _mlp_logsoftmax(idx, emb_table, W1, b1, W2T, b2):
    M, H = W2T.shape
    D = emb_table.shape[1]
    RB, NBUF, TB = _RB, _NBUF, _TB
    NB = M // RB
    NROUND = NB // NBUF
    NFULL = NB * (RB // TB)
    NT = pl.cdiv(M, TB)
    TAIL = M - NB * RB
    NR = pl.cdiv(NT, 8)
    NPAD = NR * 8
    MP = NB * RB + RB

    def body(idx_ref, emb_ref, w1_ref, b1_ref, b2_ref, w2tail_ref, w2t_hbm,
             out_ref, buf_ref, out2_ref, m_ref, s_ref, sems):
        sub = idx_ref[0] % 8
        e = emb_ref[pl.ds(sub, 1), :]
        h = lax.dot_general(e, w1_ref[...], (((1,), (0,)), ((), ())),
                            preferred_element_type=jnp.float32)
        o1 = jnp.maximum(h + b1_ref[...], 0.0)

        for rr in range(NFULL, NPAD, 8):
            out2_ref[pl.ds(rr, 8), :] = jnp.full((8, TB), -jnp.inf,
                                                 jnp.float32)

        def start(b, k):
            pltpu.make_async_copy(
                w2t_hbm.at[pl.ds(k * RB, RB), :],
                buf_ref.at[pl.ds(b * RB, RB), :],
                sems.at[b],
            ).start()

        def wait(b):
            pltpu.make_async_copy(
                w2t_hbm.at[pl.ds(0, RB), :],
                buf_ref.at[pl.ds(b * RB, RB), :],
                sems.at[b],
            ).wait()

        for b in range(NBUF):
            start(b, b)

        def stream_round(r, _):
            base = r * NBUF
            for b in range(NBUF):
                k = base + b
                wait(b)
                w = buf_ref[pl.ds(b * RB, RB), :]
                x = lax.dot_general(o1, w, (((1,), (1,)), ((), ())),
                                    preferred_element_type=jnp.float32)
                x = x + b2_ref[:, pl.ds(k * RB, RB)]
                for t in range(RB // TB):
                    out2_ref[pl.ds(k * (RB // TB) + t, 1), :] = (
                        x[:, t * TB:(t + 1) * TB])
                nxt = k + NBUF

                @pl.when(nxt < NB)
                def _():
                    start(b, nxt)

            return 0

        lax.fori_loop(0, NROUND, stream_round, 0)

        xt = lax.dot_general(o1, w2tail_ref[...], (((1,), (1,)), ((), ())),
                             preferred_element_type=jnp.float32)
        xt = xt + b2_ref[:, pl.ds(NB * RB, RB)]
        lane = lax.broadcasted_iota(jnp.int32, (1, RB), 1)
        xt = jnp.where(lane < TAIL, xt, -jnp.inf)
        for t in range(RB // TB):
            out2_ref[pl.ds(NFULL + t, 1), :] = xt[:, t * TB:(t + 1) * TB]

        m_ref[...] = jnp.full((8, 128), -jnp.inf, jnp.float32)
        s_ref[...] = jnp.zeros((8, 128), jnp.float32)

        def stats_step(j, _):
            blk = out2_ref[pl.ds(j * 8, 8), :]
            xs = blk.reshape(8, TB // 128, 128)
            bm = jnp.max(xs, axis=1)
            m_old = m_ref[...]
            m_new = jnp.maximum(m_old, bm)
            es = jnp.exp(xs - m_new[:, None, :])
            s_ref[...] = s_ref[...] * jnp.exp(m_old - m_new) + jnp.sum(
                es, axis=1)
            m_ref[...] = m_new
            return 0

        lax.fori_loop(0, NR, stats_step, 0)

        mv = m_ref[...]
        gm = jnp.max(mv)
        z = jnp.sum(s_ref[...] * jnp.exp(mv - gm))
        logz = gm + jnp.log(z)

        def emit_step(j, _):
            out_ref[pl.ds(j * 8, 8), :] = out2_ref[pl.ds(j * 8, 8), :] - logz
            return 0

        lax.fori_loop(0, NR, emit_step, 0)

    grid_spec = pltpu.PrefetchScalarGridSpec(
        num_scalar_prefetch=1,
        grid=(1,),
        in_specs=[
            pl.BlockSpec((8, D), lambda i, s: (s[0] // 8, 0)),
            pl.BlockSpec(W1.shape, lambda i, s: (0, 0)),
            pl.BlockSpec((1, H), lambda i, s: (0, 0)),
            pl.BlockSpec((1, MP), lambda i, s: (0, 0)),
            pl.BlockSpec((_RB, H), lambda i, s: (NB, 0)),
            pl.BlockSpec(memory_space=pl.ANY),
        ],
        out_specs=pl.BlockSpec((NPAD, TB), lambda i, s: (0, 0)),
        scratch_shapes=[
            pltpu.VMEM((_NBUF * _RB, H), jnp.float32),
            pltpu.VMEM((NPAD, TB), jnp.float32),
            pltpu.VMEM((8, 128), jnp.float32),
            pltpu.VMEM((8, 128), jnp.float32),
            pltpu.SemaphoreType.DMA((_NBUF,)),
        ],
    )

    out_fn = pl.pallas_call(
        body,
        grid_spec=grid_spec,
        out_shape=jax.ShapeDtypeStruct((NPAD, TB), jnp.float32),
        compiler_params=pltpu.CompilerParams(
            dimension_semantics=("arbitrary",),
        ),
    )
    b2p = jnp.pad(b2.reshape(1, M), ((0, 0), (0, MP - M)))
    out = out_fn(idx, emb_table, W1, b1.reshape(1, H), b2p, W2T, W2T)
    return out


def kernel(inputs, emb_table, W1, b1, W2, b2):
    idx = inputs.astype(jnp.int32)
    out = _mlp_logsoftmax(idx, emb_table, W1, b1, W2.T, b2)
    M = W2.shape[1]
    return out.reshape(-1)[:M].reshape(3, -1)
